# baseline (speedup 1.0000x reference)
import jax
import jax.numpy as jnp
from jax.experimental import pallas as pl
from jax.experimental.pallas import tpu as pltpu

B, T = 8, 2048
INPUT_DIM = 2
HIDDEN = 256
D = 64
K = 1024
H, P = 10, 20
N = T - H - P + 1
DP = D * P
BN = 512
NB = T // BN
CH = 512
W_REC, W_COMMIT, W_SMOOTH = 1.0, 0.25, 0.1


def _shift_rows(a, off):
    if off == 0:
        return a
    L, C = a.shape
    z = jnp.zeros((abs(off), C), a.dtype)
    if off > 0:
        return jnp.concatenate([a[off:, :], z], axis=0)
    return jnp.concatenate([z, a[:off, :]], axis=0)


def _encoder_kernel(xt_ref, w1_ref, b1_ref, w2_ref, b2_ref, ze_ref):
    xt = xt_ref[0]
    z1 = jnp.zeros((T, HIDDEN), jnp.float32)
    for k in range(7):
        z1 = z1 + jnp.dot(_shift_rows(xt, k - 3), w1_ref[k],
                          preferred_element_type=jnp.float32)
    z1 = jnp.maximum(z1 + b1_ref[...], 0.0)
    z2 = jnp.zeros((T, D), jnp.float32)
    for k in range(9):
        z2 = z2 + jnp.dot(_shift_rows(z1, k - 4), w2_ref[k],
                          preferred_element_type=jnp.float32)
    ze_ref[0] = z2 + b2_ref[...]


def _vq_kernel(ze_ref, cbt_ref, cb_ref, ct_ref,
               quant_ref, cemb_ref, codes_ref, misc_ref):
    l0 = jnp.zeros((1, D), jnp.float32)
    l1 = jnp.zeros((1, D), jnp.float32)
    idx0_chunks = []
    lane_iota = jax.lax.broadcasted_iota(jnp.int32, (CH, K), 1)

    def stage(r, q):
        cbt = cbt_ref[q]
        cnorm = jnp.sum(cbt * cbt, axis=0, keepdims=True)
        s = jnp.dot(r, cbt, preferred_element_type=jnp.float32)
        rnorm = jnp.sum(r * r, axis=1, keepdims=True)
        d2 = rnorm - 2.0 * s + cnorm
        dmin = jnp.min(d2, axis=1, keepdims=True)
        idx = jnp.min(jnp.where(d2 == dmin, lane_iota, K),
                      axis=1, keepdims=True)
        oh = (lane_iota == idx).astype(jnp.float32)
        qv = jnp.dot(oh, cb_ref[q], preferred_element_type=jnp.float32,
                     precision=jax.lax.Precision.HIGHEST)
        return idx, oh, qv

    for c in range(T // CH):
        sl = slice(c * CH, (c + 1) * CH)
        z = ze_ref[0, sl, :]
        idx0, oh0, qv0 = stage(z, 0)
        d0 = qv0 - z
        qst0 = z + d0
        r1 = z - qst0
        l0 = l0 + jnp.sum(d0 * d0, axis=0, keepdims=True)
        idx1, _, qv1 = stage(r1, 1)
        d1 = qv1 - r1
        qst1 = r1 + d1
        l1 = l1 + jnp.sum(d1 * d1, axis=0, keepdims=True)
        quant_ref[0, sl, :] = qst0 + qst1
        cemb_ref[0, sl, :] = jnp.dot(oh0, ct_ref[...],
                                     preferred_element_type=jnp.float32,
                                     precision=jax.lax.Precision.HIGHEST)
        codes_ref[0, sl, 0:1] = idx0
        codes_ref[0, sl, 1:2] = idx1
        idx0_chunks.append(idx0)

    idx0_full = jnp.concatenate(idx0_chunks, axis=0)
    tr = jnp.sum((idx0_full[1:, :] != idx0_full[:-1, :]).astype(jnp.float32),
                 axis=0, keepdims=True)
    trv = jnp.where(
        jax.lax.broadcasted_iota(jnp.int32, (1, D), 1) == 0, tr, 0.0)
    misc_ref[0, 0:1, :] = l0
    misc_ref[0, 1:2, :] = l1
    misc_ref[0, 2:3, :] = trv


def _pred_kernel(zq_ref, qt_ref, c0_ref, w3_ref, b3_ref, w4_ref, b4_ref,
                 gl_ref):
    nb = pl.program_id(1)
    n0 = nb * BN
    acc = jnp.zeros((BN, DP), jnp.float32)
    for i in range(H):
        acc = acc + jnp.dot(zq_ref[0, pl.ds(n0 + i, BN), :], w3_ref[i],
                            preferred_element_type=jnp.float32)
    h = jnp.maximum(acc + b3_ref[...], 0.0)
    zp = jnp.dot(h, w4_ref[...], preferred_element_type=jnp.float32) \
        + b4_ref[...]

    nvec = jax.lax.broadcasted_iota(jnp.int32, (BN, 1), 0) + n0
    mask = (nvec < N).astype(jnp.float32)
    ca = c0_ref[0, pl.ds(n0 + H, BN), :]
    cbn = c0_ref[0, pl.ds(n0 + H + 1, BN), :]
    sc = (ca == cbn).astype(jnp.float32)
    total = jnp.zeros((BN, 1), jnp.float32)
    for p in range(P):
        zt = qt_ref[0, pl.ds(n0 + H + p, BN), :]
        zpp = zp[:, p * D:(p + 1) * D]
        num = jnp.sum(zt * zpp, axis=1, keepdims=True)
        nt = jnp.sqrt(jnp.sum(zt * zt, axis=1, keepdims=True))
        npp = jnp.sqrt(jnp.sum(zpp * zpp, axis=1, keepdims=True))
        den = jnp.maximum(nt, 1e-8) * jnp.maximum(npp, 1e-8)
        sim = num / den
        dlt = sim - sc
        total = total + dlt * dlt
    tsum = jnp.sum(total * mask)
    gl_ref[0, 0] = jnp.where(
        jax.lax.broadcasted_iota(jnp.int32, (1, 128), 1) == 0, tsum, 0.0)


def kernel(traj_xy, masks, W1, b1, W2, b2, W3, b3, W4, b4, code_table,
           codebooks):
    f32 = jnp.float32
    xt = jnp.transpose(traj_xy, (0, 2, 1))
    w1 = jnp.transpose(W1, (2, 1, 0))
    w2 = jnp.transpose(W2, (2, 1, 0))

    ze = pl.pallas_call(
        _encoder_kernel,
        grid=(B,),
        in_specs=[
            pl.BlockSpec((1, T, INPUT_DIM), lambda b: (b, 0, 0)),
            pl.BlockSpec((7, INPUT_DIM, HIDDEN), lambda b: (0, 0, 0)),
            pl.BlockSpec((1, HIDDEN), lambda b: (0, 0)),
            pl.BlockSpec((9, HIDDEN, D), lambda b: (0, 0, 0)),
            pl.BlockSpec((1, D), lambda b: (0, 0)),
        ],
        out_specs=pl.BlockSpec((1, T, D), lambda b: (b, 0, 0)),
        out_shape=jax.ShapeDtypeStruct((B, T, D), f32),
        compiler_params=pltpu.CompilerParams(
            dimension_semantics=("parallel",)),
    )(xt, w1, b1.reshape(1, HIDDEN), w2, b2.reshape(1, D))

    cbt = jnp.transpose(codebooks, (0, 2, 1))
    quant, cemb, codes, misc = pl.pallas_call(
        _vq_kernel,
        grid=(B,),
        in_specs=[
            pl.BlockSpec((1, T, D), lambda b: (b, 0, 0)),
            pl.BlockSpec((2, D, K), lambda b: (0, 0, 0)),
            pl.BlockSpec((2, K, D), lambda b: (0, 0, 0)),
            pl.BlockSpec((K, D), lambda b: (0, 0)),
        ],
        out_specs=[
            pl.BlockSpec((1, T, D), lambda b: (b, 0, 0)),
            pl.BlockSpec((1, T, D), lambda b: (b, 0, 0)),
            pl.BlockSpec((1, T, 2), lambda b: (b, 0, 0)),
            pl.BlockSpec((1, 3, D), lambda b: (b, 0, 0)),
        ],
        out_shape=[
            jax.ShapeDtypeStruct((B, T, D), f32),
            jax.ShapeDtypeStruct((B, T, D), f32),
            jax.ShapeDtypeStruct((B, T, 2), jnp.int32),
            jax.ShapeDtypeStruct((B, 3, D), f32),
        ],
        compiler_params=pltpu.CompilerParams(
            dimension_semantics=("parallel",)),
    )(ze, cbt, codebooks, code_table)

    zq = jnp.concatenate([quant, cemb], axis=-1)
    zqp = jnp.pad(zq, ((0, 0), (0, 16), (0, 0)))
    qtp = jnp.pad(quant, ((0, 0), (0, 32), (0, 0)))
    c0p = jnp.pad(codes[:, :, 0:1], ((0, 0), (0, 32), (0, 0)))
    w3r = jnp.transpose(W3.reshape(DP, H, 2 * D), (1, 2, 0))
    w4t = W4.T

    gl = pl.pallas_call(
        _pred_kernel,
        grid=(B, NB),
        in_specs=[
            pl.BlockSpec((1, T + 16, 2 * D), lambda b, nb: (b, 0, 0)),
            pl.BlockSpec((1, T + 32, D), lambda b, nb: (b, 0, 0)),
            pl.BlockSpec((1, T + 32, 1), lambda b, nb: (b, 0, 0)),
            pl.BlockSpec((H, 2 * D, DP), lambda b, nb: (0, 0, 0)),
            pl.BlockSpec((1, DP), lambda b, nb: (0, 0)),
            pl.BlockSpec((DP, DP), lambda b, nb: (0, 0)),
            pl.BlockSpec((1, DP), lambda b, nb: (0, 0)),
        ],
        out_specs=pl.BlockSpec((1, 1, 1, 128), lambda b, nb: (b, nb, 0, 0)),
        out_shape=jax.ShapeDtypeStruct((B, NB, 1, 128), f32),
        compiler_params=pltpu.CompilerParams(
            dimension_semantics=("parallel", "parallel")),
    )(zqp, qtp, c0p, w3r, b3.reshape(1, DP), w4t, b4.reshape(1, DP))

    group_loss = jnp.sum(gl) / (B * N * P)
    vq0 = jnp.sum(misc[:, 0, :]) / (B * T * D)
    vq1 = jnp.sum(misc[:, 1, :]) / (B * T * D)
    vq_loss = 0.5 * (vq0 + vq1)
    smooth_loss = jnp.sum(misc[:, 2, :]) / (B * (T - 1))
    loss = group_loss * W_REC + vq_loss * W_COMMIT + smooth_loss * W_SMOOTH
    return codes, loss, group_loss, vq_loss, smooth_loss


# speedup vs baseline: 1.1370x; 1.1370x over previous
import jax
import jax.numpy as jnp
from jax.experimental import pallas as pl
from jax.experimental.pallas import tpu as pltpu

B, T = 8, 2048
INPUT_DIM = 2
HIDDEN = 256
D = 64
K = 1024
H, P = 10, 20
N = T - H - P + 1
DP = D * P
BN = 512
NB = T // BN
CH = 512
W_REC, W_COMMIT, W_SMOOTH = 1.0, 0.25, 0.1


def _shift_rows(a, off):
    if off == 0:
        return a
    L, C = a.shape
    z = jnp.zeros((abs(off), C), a.dtype)
    if off > 0:
        return jnp.concatenate([a[off:, :], z], axis=0)
    return jnp.concatenate([z, a[:off, :]], axis=0)


def _encoder_kernel(xt_ref, w1_ref, b1_ref, w2_ref, b2_ref, ze_ref):
    xt = xt_ref[0]
    z1 = jnp.zeros((T, HIDDEN), jnp.float32)
    for k in range(7):
        z1 = z1 + jnp.dot(_shift_rows(xt, k - 3), w1_ref[k],
                          preferred_element_type=jnp.float32)
    z1 = jnp.maximum(z1 + b1_ref[...], 0.0)
    z2 = jnp.zeros((T, D), jnp.float32)
    for k in range(9):
        z2 = z2 + jnp.dot(_shift_rows(z1, k - 4), w2_ref[k],
                          preferred_element_type=jnp.float32)
    ze_ref[0] = z2 + b2_ref[...]


def _split3(x):
    hi = x.astype(jnp.bfloat16)
    r1 = x - hi.astype(jnp.float32)
    mid = r1.astype(jnp.bfloat16)
    lo = (r1 - mid.astype(jnp.float32)).astype(jnp.bfloat16)
    return (hi, mid, lo)


def _vq_kernel(ze_ref, cbt_ref, cb_ref, ct_ref,
               quant_ref, cemb_ref, codes_ref, misc_ref):
    l0 = jnp.zeros((1, D), jnp.float32)
    l1 = jnp.zeros((1, D), jnp.float32)
    idx0_chunks = []
    lane_iota = jax.lax.broadcasted_iota(jnp.int32, (CH, K), 1)

    cb_parts = (_split3(cb_ref[0]), _split3(cb_ref[1]))
    ct_parts = _split3(ct_ref[...])

    def lookup(oh, parts):
        qv = jnp.zeros((CH, D), jnp.float32)
        for p in range(3):
            qv = qv + jnp.dot(oh, parts[p],
                              preferred_element_type=jnp.float32)
        return qv

    def stage(r, q):
        cbt = cbt_ref[q]
        cnorm = jnp.sum(cbt * cbt, axis=0, keepdims=True)
        s = jnp.dot(r, cbt, preferred_element_type=jnp.float32)
        rnorm = jnp.sum(r * r, axis=1, keepdims=True)
        d2 = rnorm - 2.0 * s + cnorm
        dmin = jnp.min(d2, axis=1, keepdims=True)
        idx = jnp.min(jnp.where(d2 == dmin, lane_iota, K),
                      axis=1, keepdims=True)
        oh = (lane_iota == idx).astype(jnp.bfloat16)
        qv = lookup(oh, cb_parts[q])
        return idx, oh, qv

    for c in range(T // CH):
        sl = slice(c * CH, (c + 1) * CH)
        z = ze_ref[0, sl, :]
        idx0, oh0, qv0 = stage(z, 0)
        d0 = qv0 - z
        qst0 = z + d0
        r1 = z - qst0
        l0 = l0 + jnp.sum(d0 * d0, axis=0, keepdims=True)
        idx1, _, qv1 = stage(r1, 1)
        d1 = qv1 - r1
        qst1 = r1 + d1
        l1 = l1 + jnp.sum(d1 * d1, axis=0, keepdims=True)
        quant_ref[0, sl, :] = qst0 + qst1
        cemb_ref[0, sl, :] = lookup(oh0, ct_parts)
        codes_ref[0, sl, 0:1] = idx0
        codes_ref[0, sl, 1:2] = idx1
        idx0_chunks.append(idx0)

    idx0_full = jnp.concatenate(idx0_chunks, axis=0)
    tr = jnp.sum((idx0_full[1:, :] != idx0_full[:-1, :]).astype(jnp.float32),
                 axis=0, keepdims=True)
    trv = jnp.where(
        jax.lax.broadcasted_iota(jnp.int32, (1, D), 1) == 0, tr, 0.0)
    misc_ref[0, 0:1, :] = l0
    misc_ref[0, 1:2, :] = l1
    misc_ref[0, 2:3, :] = trv


def _pred_kernel(zq_ref, qt_ref, c0_ref, w3_ref, b3_ref, w4_ref, b4_ref,
                 gl_ref):
    nb = pl.program_id(1)
    n0 = nb * BN
    acc = jnp.zeros((BN, DP), jnp.float32)
    for i in range(H):
        acc = acc + jnp.dot(zq_ref[0, pl.ds(n0 + i, BN), :], w3_ref[i],
                            preferred_element_type=jnp.float32)
    h = jnp.maximum(acc + b3_ref[...], 0.0)
    zp = jnp.dot(h, w4_ref[...], preferred_element_type=jnp.float32) \
        + b4_ref[...]

    nvec = jax.lax.broadcasted_iota(jnp.int32, (BN, 1), 0) + n0
    mask = (nvec < N).astype(jnp.float32)
    ca = c0_ref[0, pl.ds(n0 + H, BN), :]
    cbn = c0_ref[0, pl.ds(n0 + H + 1, BN), :]
    sc = (ca == cbn).astype(jnp.float32)
    total = jnp.zeros((BN, 1), jnp.float32)
    for p in range(P):
        zt = qt_ref[0, pl.ds(n0 + H + p, BN), :]
        zpp = zp[:, p * D:(p + 1) * D]
        num = jnp.sum(zt * zpp, axis=1, keepdims=True)
        nt = jnp.sqrt(jnp.sum(zt * zt, axis=1, keepdims=True))
        npp = jnp.sqrt(jnp.sum(zpp * zpp, axis=1, keepdims=True))
        den = jnp.maximum(nt, 1e-8) * jnp.maximum(npp, 1e-8)
        sim = num / den
        dlt = sim - sc
        total = total + dlt * dlt
    tsum = jnp.sum(total * mask)
    gl_ref[0, 0] = jnp.where(
        jax.lax.broadcasted_iota(jnp.int32, (1, 128), 1) == 0, tsum, 0.0)


def kernel(traj_xy, masks, W1, b1, W2, b2, W3, b3, W4, b4, code_table,
           codebooks):
    f32 = jnp.float32
    xt = jnp.transpose(traj_xy, (0, 2, 1))
    w1 = jnp.transpose(W1, (2, 1, 0))
    w2 = jnp.transpose(W2, (2, 1, 0))

    ze = pl.pallas_call(
        _encoder_kernel,
        grid=(B,),
        in_specs=[
            pl.BlockSpec((1, T, INPUT_DIM), lambda b: (b, 0, 0)),
            pl.BlockSpec((7, INPUT_DIM, HIDDEN), lambda b: (0, 0, 0)),
            pl.BlockSpec((1, HIDDEN), lambda b: (0, 0)),
            pl.BlockSpec((9, HIDDEN, D), lambda b: (0, 0, 0)),
            pl.BlockSpec((1, D), lambda b: (0, 0)),
        ],
        out_specs=pl.BlockSpec((1, T, D), lambda b: (b, 0, 0)),
        out_shape=jax.ShapeDtypeStruct((B, T, D), f32),
        compiler_params=pltpu.CompilerParams(
            dimension_semantics=("parallel",)),
    )(xt, w1, b1.reshape(1, HIDDEN), w2, b2.reshape(1, D))

    cbt = jnp.transpose(codebooks, (0, 2, 1))
    quant, cemb, codes, misc = pl.pallas_call(
        _vq_kernel,
        grid=(B,),
        in_specs=[
            pl.BlockSpec((1, T, D), lambda b: (b, 0, 0)),
            pl.BlockSpec((2, D, K), lambda b: (0, 0, 0)),
            pl.BlockSpec((2, K, D), lambda b: (0, 0, 0)),
            pl.BlockSpec((K, D), lambda b: (0, 0)),
        ],
        out_specs=[
            pl.BlockSpec((1, T, D), lambda b: (b, 0, 0)),
            pl.BlockSpec((1, T, D), lambda b: (b, 0, 0)),
            pl.BlockSpec((1, T, 2), lambda b: (b, 0, 0)),
            pl.BlockSpec((1, 3, D), lambda b: (b, 0, 0)),
        ],
        out_shape=[
            jax.ShapeDtypeStruct((B, T, D), f32),
            jax.ShapeDtypeStruct((B, T, D), f32),
            jax.ShapeDtypeStruct((B, T, 2), jnp.int32),
            jax.ShapeDtypeStruct((B, 3, D), f32),
        ],
        compiler_params=pltpu.CompilerParams(
            dimension_semantics=("parallel",)),
    )(ze, cbt, codebooks, code_table)

    zq = jnp.concatenate([quant, cemb], axis=-1)
    zqp = jnp.pad(zq, ((0, 0), (0, 16), (0, 0)))
    qtp = jnp.pad(quant, ((0, 0), (0, 32), (0, 0)))
    c0p = jnp.pad(codes[:, :, 0:1], ((0, 0), (0, 32), (0, 0)))
    w3r = jnp.transpose(W3.reshape(DP, H, 2 * D), (1, 2, 0))
    w4t = W4.T

    gl = pl.pallas_call(
        _pred_kernel,
        grid=(B, NB),
        in_specs=[
            pl.BlockSpec((1, T + 16, 2 * D), lambda b, nb: (b, 0, 0)),
            pl.BlockSpec((1, T + 32, D), lambda b, nb: (b, 0, 0)),
            pl.BlockSpec((1, T + 32, 1), lambda b, nb: (b, 0, 0)),
            pl.BlockSpec((H, 2 * D, DP), lambda b, nb: (0, 0, 0)),
            pl.BlockSpec((1, DP), lambda b, nb: (0, 0)),
            pl.BlockSpec((DP, DP), lambda b, nb: (0, 0)),
            pl.BlockSpec((1, DP), lambda b, nb: (0, 0)),
        ],
        out_specs=pl.BlockSpec((1, 1, 1, 128), lambda b, nb: (b, nb, 0, 0)),
        out_shape=jax.ShapeDtypeStruct((B, NB, 1, 128), f32),
        compiler_params=pltpu.CompilerParams(
            dimension_semantics=("parallel", "parallel")),
    )(zqp, qtp, c0p, w3r, b3.reshape(1, DP), w4t, b4.reshape(1, DP))

    group_loss = jnp.sum(gl) / (B * N * P)
    vq0 = jnp.sum(misc[:, 0, :]) / (B * T * D)
    vq1 = jnp.sum(misc[:, 1, :]) / (B * T * D)
    vq_loss = 0.5 * (vq0 + vq1)
    smooth_loss = jnp.sum(misc[:, 2, :]) / (B * (T - 1))
    loss = group_loss * W_REC + vq_loss * W_COMMIT + smooth_loss * W_SMOOTH
    return codes, loss, group_loss, vq_loss, smooth_loss


# speedup vs baseline: 1.2366x; 1.0875x over previous
import jax
import jax.numpy as jnp
from jax.experimental import pallas as pl
from jax.experimental.pallas import tpu as pltpu

B, T = 8, 2048
INPUT_DIM = 2
HIDDEN = 256
D = 64
K = 1024
H, P = 10, 20
N = T - H - P + 1
DP = D * P
BN = 512
NB = T // BN
CH = 512
W_REC, W_COMMIT, W_SMOOTH = 1.0, 0.25, 0.1


def _shift_rows(a, off):
    if off == 0:
        return a
    L, C = a.shape
    z = jnp.zeros((abs(off), C), a.dtype)
    if off > 0:
        return jnp.concatenate([a[off:, :], z], axis=0)
    return jnp.concatenate([z, a[:off, :]], axis=0)


def _encoder_kernel(xt_ref, w1_ref, b1_ref, w2_ref, b2_ref, ze_ref):
    xt = xt_ref[0]
    z1 = jnp.zeros((T, HIDDEN), jnp.float32)
    for k in range(7):
        z1 = z1 + jnp.dot(_shift_rows(xt, k - 3), w1_ref[k],
                          preferred_element_type=jnp.float32)
    z1 = jnp.maximum(z1 + b1_ref[...], 0.0)
    z2 = jnp.zeros((T, D), jnp.float32)
    for k in range(9):
        z2 = z2 + jnp.dot(_shift_rows(z1, k - 4), w2_ref[k],
                          preferred_element_type=jnp.float32)
    ze_ref[0] = z2 + b2_ref[...]


def _split3(x):
    hi = x.astype(jnp.bfloat16)
    r1 = x - hi.astype(jnp.float32)
    mid = r1.astype(jnp.bfloat16)
    lo = (r1 - mid.astype(jnp.float32)).astype(jnp.bfloat16)
    return (hi, mid, lo)


def _vq_kernel(ze_ref, cbt_ref, cb_ref, ct_ref,
               quant_ref, cemb_ref, codes_ref, misc_ref):
    l0 = jnp.zeros((1, D), jnp.float32)
    l1 = jnp.zeros((1, D), jnp.float32)
    idx0_chunks = []
    lane_iota = jax.lax.broadcasted_iota(jnp.int32, (CH, K), 1)

    cb0h, cb0m, cb0l = _split3(cb_ref[0])
    cb1h, cb1m, cb1l = _split3(cb_ref[1])
    cth, ctm, ctl = _split3(ct_ref[...])
    t0a = jnp.concatenate([cb0h, cb0m], axis=1)
    t0b = jnp.concatenate([cb0l, cth], axis=1)
    t0c = jnp.concatenate([ctm, ctl], axis=1)
    t1a = jnp.concatenate([cb1h, cb1m], axis=1)

    cbt0, cbt1 = cbt_ref[0], cbt_ref[1]
    cnorm0 = jnp.sum(cbt0 * cbt0, axis=0, keepdims=True)
    cnorm1 = jnp.sum(cbt1 * cbt1, axis=0, keepdims=True)

    def stage(r, cbt, cnorm):
        s = jnp.dot(r, cbt, preferred_element_type=jnp.float32)
        rnorm = jnp.sum(r * r, axis=1, keepdims=True)
        d2 = rnorm - 2.0 * s + cnorm
        dmin = jnp.min(d2, axis=1, keepdims=True)
        idx = jnp.min(jnp.where(d2 == dmin, lane_iota, K),
                      axis=1, keepdims=True)
        oh = (lane_iota == idx).astype(jnp.bfloat16)
        return idx, oh

    def dotf(oh, tab):
        return jnp.dot(oh, tab, preferred_element_type=jnp.float32)

    for c in range(T // CH):
        sl = slice(c * CH, (c + 1) * CH)
        z = ze_ref[0, sl, :]
        idx0, oh0 = stage(z, cbt0, cnorm0)
        p1 = dotf(oh0, t0a)
        p2 = dotf(oh0, t0b)
        p3 = dotf(oh0, t0c)
        qv0 = (p1[:, :D] + p1[:, D:]) + p2[:, :D]
        d0 = qv0 - z
        qst0 = z + d0
        r1 = z - qst0
        l0 = l0 + jnp.sum(d0 * d0, axis=0, keepdims=True)
        idx1, oh1 = stage(r1, cbt1, cnorm1)
        q1 = dotf(oh1, t1a)
        qv1 = (q1[:, :D] + q1[:, D:]) + dotf(oh1, cb1l)
        d1 = qv1 - r1
        qst1 = r1 + d1
        l1 = l1 + jnp.sum(d1 * d1, axis=0, keepdims=True)
        quant_ref[0, sl, :] = qst0 + qst1
        cemb_ref[0, sl, :] = (p2[:, D:] + p3[:, :D]) + p3[:, D:]
        codes_ref[0, sl, 0:1] = idx0
        codes_ref[0, sl, 1:2] = idx1
        idx0_chunks.append(idx0)

    idx0_full = jnp.concatenate(idx0_chunks, axis=0)
    tr = jnp.sum((idx0_full[1:, :] != idx0_full[:-1, :]).astype(jnp.float32),
                 axis=0, keepdims=True)
    trv = jnp.where(
        jax.lax.broadcasted_iota(jnp.int32, (1, D), 1) == 0, tr, 0.0)
    misc_ref[0, 0:1, :] = l0
    misc_ref[0, 1:2, :] = l1
    misc_ref[0, 2:3, :] = trv


def _pred_kernel(zq_ref, qt_ref, c0_ref, w3_ref, b3_ref, w4_ref, b4_ref,
                 gl_ref):
    nb = pl.program_id(1)
    n0 = nb * BN
    acc = jnp.zeros((BN, DP), jnp.float32)
    for i in range(H):
        acc = acc + jnp.dot(zq_ref[0, pl.ds(n0 + i, BN), :], w3_ref[i],
                            preferred_element_type=jnp.float32)
    h = jnp.maximum(acc + b3_ref[...], 0.0)
    zp = jnp.dot(h, w4_ref[...], preferred_element_type=jnp.float32) \
        + b4_ref[...]

    nvec = jax.lax.broadcasted_iota(jnp.int32, (BN, 1), 0) + n0
    mask = (nvec < N).astype(jnp.float32)
    ca = c0_ref[0, pl.ds(n0 + H, BN), :]
    cbn = c0_ref[0, pl.ds(n0 + H + 1, BN), :]
    sc = (ca == cbn).astype(jnp.float32)
    total = jnp.zeros((BN, 1), jnp.float32)
    for p in range(P):
        zt = qt_ref[0, pl.ds(n0 + H + p, BN), :]
        zpp = zp[:, p * D:(p + 1) * D]
        num = jnp.sum(zt * zpp, axis=1, keepdims=True)
        nt = jnp.sqrt(jnp.sum(zt * zt, axis=1, keepdims=True))
        npp = jnp.sqrt(jnp.sum(zpp * zpp, axis=1, keepdims=True))
        den = jnp.maximum(nt, 1e-8) * jnp.maximum(npp, 1e-8)
        sim = num / den
        dlt = sim - sc
        total = total + dlt * dlt
    tsum = jnp.sum(total * mask)
    gl_ref[0, 0] = jnp.where(
        jax.lax.broadcasted_iota(jnp.int32, (1, 128), 1) == 0, tsum, 0.0)


def kernel(traj_xy, masks, W1, b1, W2, b2, W3, b3, W4, b4, code_table,
           codebooks):
    f32 = jnp.float32
    xt = jnp.transpose(traj_xy, (0, 2, 1))
    w1 = jnp.transpose(W1, (2, 1, 0))
    w2 = jnp.transpose(W2, (2, 1, 0))

    ze = pl.pallas_call(
        _encoder_kernel,
        grid=(B,),
        in_specs=[
            pl.BlockSpec((1, T, INPUT_DIM), lambda b: (b, 0, 0)),
            pl.BlockSpec((7, INPUT_DIM, HIDDEN), lambda b: (0, 0, 0)),
            pl.BlockSpec((1, HIDDEN), lambda b: (0, 0)),
            pl.BlockSpec((9, HIDDEN, D), lambda b: (0, 0, 0)),
            pl.BlockSpec((1, D), lambda b: (0, 0)),
        ],
        out_specs=pl.BlockSpec((1, T, D), lambda b: (b, 0, 0)),
        out_shape=jax.ShapeDtypeStruct((B, T, D), f32),
        compiler_params=pltpu.CompilerParams(
            dimension_semantics=("parallel",)),
    )(xt, w1, b1.reshape(1, HIDDEN), w2, b2.reshape(1, D))

    cbt = jnp.transpose(codebooks, (0, 2, 1))
    quant, cemb, codes, misc = pl.pallas_call(
        _vq_kernel,
        grid=(B,),
        in_specs=[
            pl.BlockSpec((1, T, D), lambda b: (b, 0, 0)),
            pl.BlockSpec((2, D, K), lambda b: (0, 0, 0)),
            pl.BlockSpec((2, K, D), lambda b: (0, 0, 0)),
            pl.BlockSpec((K, D), lambda b: (0, 0)),
        ],
        out_specs=[
            pl.BlockSpec((1, T, D), lambda b: (b, 0, 0)),
            pl.BlockSpec((1, T, D), lambda b: (b, 0, 0)),
            pl.BlockSpec((1, T, 2), lambda b: (b, 0, 0)),
            pl.BlockSpec((1, 3, D), lambda b: (b, 0, 0)),
        ],
        out_shape=[
            jax.ShapeDtypeStruct((B, T, D), f32),
            jax.ShapeDtypeStruct((B, T, D), f32),
            jax.ShapeDtypeStruct((B, T, 2), jnp.int32),
            jax.ShapeDtypeStruct((B, 3, D), f32),
        ],
        compiler_params=pltpu.CompilerParams(
            dimension_semantics=("parallel",)),
    )(ze, cbt, codebooks, code_table)

    zq = jnp.concatenate([quant, cemb], axis=-1)
    zqp = jnp.pad(zq, ((0, 0), (0, 16), (0, 0)))
    qtp = jnp.pad(quant, ((0, 0), (0, 32), (0, 0)))
    c0p = jnp.pad(codes[:, :, 0:1], ((0, 0), (0, 32), (0, 0)))
    w3r = jnp.transpose(W3.reshape(DP, H, 2 * D), (1, 2, 0))
    w4t = W4.T

    gl = pl.pallas_call(
        _pred_kernel,
        grid=(B, NB),
        in_specs=[
            pl.BlockSpec((1, T + 16, 2 * D), lambda b, nb: (b, 0, 0)),
            pl.BlockSpec((1, T + 32, D), lambda b, nb: (b, 0, 0)),
            pl.BlockSpec((1, T + 32, 1), lambda b, nb: (b, 0, 0)),
            pl.BlockSpec((H, 2 * D, DP), lambda b, nb: (0, 0, 0)),
            pl.BlockSpec((1, DP), lambda b, nb: (0, 0)),
            pl.BlockSpec((DP, DP), lambda b, nb: (0, 0)),
            pl.BlockSpec((1, DP), lambda b, nb: (0, 0)),
        ],
        out_specs=pl.BlockSpec((1, 1, 1, 128), lambda b, nb: (b, nb, 0, 0)),
        out_shape=jax.ShapeDtypeStruct((B, NB, 1, 128), f32),
        compiler_params=pltpu.CompilerParams(
            dimension_semantics=("parallel", "parallel")),
    )(zqp, qtp, c0p, w3r, b3.reshape(1, DP), w4t, b4.reshape(1, DP))

    group_loss = jnp.sum(gl) / (B * N * P)
    vq0 = jnp.sum(misc[:, 0, :]) / (B * T * D)
    vq1 = jnp.sum(misc[:, 1, :]) / (B * T * D)
    vq_loss = 0.5 * (vq0 + vq1)
    smooth_loss = jnp.sum(misc[:, 2, :]) / (B * (T - 1))
    loss = group_loss * W_REC + vq_loss * W_COMMIT + smooth_loss * W_SMOOTH
    return codes, loss, group_loss, vq_loss, smooth_loss


# speedup vs baseline: 1.2603x; 1.0192x over previous
import jax
import jax.numpy as jnp
from jax.experimental import pallas as pl
from jax.experimental.pallas import tpu as pltpu

B, T = 8, 2048
INPUT_DIM = 2
HIDDEN = 256
D = 64
K = 1024
H, P = 10, 20
N = T - H - P + 1
DP = D * P
BN = 512
NB = T // BN
CH = 512
W_REC, W_COMMIT, W_SMOOTH = 1.0, 0.25, 0.1


def _shift_rows(a, off):
    if off == 0:
        return a
    L, C = a.shape
    z = jnp.zeros((abs(off), C), a.dtype)
    if off > 0:
        return jnp.concatenate([a[off:, :], z], axis=0)
    return jnp.concatenate([z, a[:off, :]], axis=0)


def _encoder_kernel(xt_ref, w1_ref, b1_ref, w2_ref, b2_ref, ze_ref):
    xt = xt_ref[0]
    z1 = jnp.zeros((T, HIDDEN), jnp.float32)
    for k in range(7):
        z1 = z1 + jnp.dot(_shift_rows(xt, k - 3), w1_ref[k],
                          preferred_element_type=jnp.float32)
    z1 = jnp.maximum(z1 + b1_ref[...], 0.0)
    z2 = jnp.zeros((T, D), jnp.float32)
    for k in range(9):
        z2 = z2 + jnp.dot(_shift_rows(z1, k - 4), w2_ref[k],
                          preferred_element_type=jnp.float32)
    ze_ref[0] = z2 + b2_ref[...]


def _split3(x):
    hi = x.astype(jnp.bfloat16)
    r1 = x - hi.astype(jnp.float32)
    mid = r1.astype(jnp.bfloat16)
    lo = (r1 - mid.astype(jnp.float32)).astype(jnp.bfloat16)
    return (hi, mid, lo)


def _vq_kernel(ze_ref, cbt_ref, cb_ref, ct_ref,
               quant_ref, cemb_ref, codes_ref, misc_ref):
    l0 = jnp.zeros((1, D), jnp.float32)
    l1 = jnp.zeros((1, D), jnp.float32)
    idx0_chunks = []
    lane_iota = jax.lax.broadcasted_iota(jnp.int32, (CH, K), 1)

    cb0h, cb0m, cb0l = _split3(cb_ref[0])
    cb1h, cb1m, cb1l = _split3(cb_ref[1])
    cth, ctm, ctl = _split3(ct_ref[...])
    t0a = jnp.concatenate([cb0h, cb0m], axis=1)
    t0b = jnp.concatenate([cb0l, cth], axis=1)
    t0c = jnp.concatenate([ctm, ctl], axis=1)
    t1a = jnp.concatenate([cb1h, cb1m], axis=1)

    cbt0, cbt1 = cbt_ref[0], cbt_ref[1]
    cnorm0 = jnp.sum(cbt0 * cbt0, axis=0, keepdims=True)
    cnorm1 = jnp.sum(cbt1 * cbt1, axis=0, keepdims=True)

    def stage(r, cbt, cnorm):
        s = jnp.dot(r, cbt, preferred_element_type=jnp.float32)
        rnorm = jnp.sum(r * r, axis=1, keepdims=True)
        d2 = rnorm - 2.0 * s + cnorm
        dmin = jnp.min(d2, axis=1, keepdims=True)
        idx = jnp.min(jnp.where(d2 == dmin, lane_iota, K),
                      axis=1, keepdims=True)
        oh = (lane_iota == idx).astype(jnp.bfloat16)
        return idx, oh

    def dotf(oh, tab):
        return jnp.dot(oh, tab, preferred_element_type=jnp.float32)

    for c in range(T // CH):
        sl = slice(c * CH, (c + 1) * CH)
        z = ze_ref[0, sl, :]
        idx0, oh0 = stage(z, cbt0, cnorm0)
        p1 = dotf(oh0, t0a)
        p2 = dotf(oh0, t0b)
        p3 = dotf(oh0, t0c)
        qv0 = (p1[:, :D] + p1[:, D:]) + p2[:, :D]
        d0 = qv0 - z
        qst0 = z + d0
        r1 = z - qst0
        l0 = l0 + jnp.sum(d0 * d0, axis=0, keepdims=True)
        idx1, oh1 = stage(r1, cbt1, cnorm1)
        q1 = dotf(oh1, t1a)
        qv1 = (q1[:, :D] + q1[:, D:]) + dotf(oh1, cb1l)
        d1 = qv1 - r1
        qst1 = r1 + d1
        l1 = l1 + jnp.sum(d1 * d1, axis=0, keepdims=True)
        quant_ref[0, sl, :] = qst0 + qst1
        cemb_ref[0, sl, :] = (p2[:, D:] + p3[:, :D]) + p3[:, D:]
        codes_ref[0, sl, 0:1] = idx0
        codes_ref[0, sl, 1:2] = idx1
        idx0_chunks.append(idx0)

    idx0_full = jnp.concatenate(idx0_chunks, axis=0)
    tr = jnp.sum((idx0_full[1:, :] != idx0_full[:-1, :]).astype(jnp.float32),
                 axis=0, keepdims=True)
    trv = jnp.where(
        jax.lax.broadcasted_iota(jnp.int32, (1, D), 1) == 0, tr, 0.0)
    misc_ref[0, 0:1, :] = l0
    misc_ref[0, 1:2, :] = l1
    misc_ref[0, 2:3, :] = trv


def _pred_kernel(zq_ref, qt_ref, c0_ref, w3_ref, b3_ref, w4_ref, b4_ref,
                 gl_ref):
    nb = pl.program_id(1)
    n0 = nb * BN
    acc = jnp.zeros((BN, DP), jnp.float32)
    for i in range(H):
        acc = acc + jnp.dot(zq_ref[0, pl.ds(n0 + i, BN), :], w3_ref[i],
                            preferred_element_type=jnp.float32)
    h = jnp.maximum(acc + b3_ref[...], 0.0)
    zp = jnp.dot(h, w4_ref[...], preferred_element_type=jnp.float32) \
        + b4_ref[...]

    nvec = jax.lax.broadcasted_iota(jnp.int32, (BN, 1), 0) + n0
    mask = (nvec < N).astype(jnp.float32)
    ca = c0_ref[0, pl.ds(n0 + H, BN), :]
    cbn = c0_ref[0, pl.ds(n0 + H + 1, BN), :]
    sc = (ca == cbn).astype(jnp.float32)
    total = jnp.zeros((BN, 1), jnp.float32)
    for p in range(P):
        zt = qt_ref[0, pl.ds(n0 + H + p, BN), :]
        zpp = zp[:, p * D:(p + 1) * D]
        num = jnp.sum(zt * zpp, axis=1, keepdims=True)
        nt = jnp.sqrt(jnp.sum(zt * zt, axis=1, keepdims=True))
        npp = jnp.sqrt(jnp.sum(zpp * zpp, axis=1, keepdims=True))
        den = jnp.maximum(nt, 1e-8) * jnp.maximum(npp, 1e-8)
        sim = num / den
        dlt = sim - sc
        total = total + dlt * dlt
    tsum = jnp.sum(total * mask)
    gl_ref[0, 0] = jnp.where(
        jax.lax.broadcasted_iota(jnp.int32, (1, 128), 1) == 0, tsum, 0.0)


def kernel(traj_xy, masks, W1, b1, W2, b2, W3, b3, W4, b4, code_table,
           codebooks):
    f32 = jnp.float32
    xt = jnp.transpose(traj_xy, (0, 2, 1))
    w1 = jnp.transpose(W1, (2, 1, 0))
    w2 = jnp.transpose(W2, (2, 1, 0))

    ze = pl.pallas_call(
        _encoder_kernel,
        grid=(B,),
        in_specs=[
            pl.BlockSpec((1, T, INPUT_DIM), lambda b: (b, 0, 0)),
            pl.BlockSpec((7, INPUT_DIM, HIDDEN), lambda b: (0, 0, 0)),
            pl.BlockSpec((1, HIDDEN), lambda b: (0, 0)),
            pl.BlockSpec((9, HIDDEN, D), lambda b: (0, 0, 0)),
            pl.BlockSpec((1, D), lambda b: (0, 0)),
        ],
        out_specs=pl.BlockSpec((1, T, D), lambda b: (b, 0, 0)),
        out_shape=jax.ShapeDtypeStruct((B, T, D), f32),
        compiler_params=pltpu.CompilerParams(
            dimension_semantics=("parallel",)),
    )(xt, w1, b1.reshape(1, HIDDEN), w2, b2.reshape(1, D))

    cbt = jnp.transpose(codebooks, (0, 2, 1))
    quant, cemb, codes, misc = pl.pallas_call(
        _vq_kernel,
        grid=(B,),
        in_specs=[
            pl.BlockSpec((1, T, D), lambda b: (b, 0, 0)),
            pl.BlockSpec((2, D, K), lambda b: (0, 0, 0)),
            pl.BlockSpec((2, K, D), lambda b: (0, 0, 0)),
            pl.BlockSpec((K, D), lambda b: (0, 0)),
        ],
        out_specs=[
            pl.BlockSpec((1, T, D), lambda b: (b, 0, 0)),
            pl.BlockSpec((1, T, D), lambda b: (b, 0, 0)),
            pl.BlockSpec((1, T, 2), lambda b: (b, 0, 0)),
            pl.BlockSpec((1, 3, D), lambda b: (b, 0, 0)),
        ],
        out_shape=[
            jax.ShapeDtypeStruct((B, T, D), f32),
            jax.ShapeDtypeStruct((B, T, D), f32),
            jax.ShapeDtypeStruct((B, T, 2), jnp.int32),
            jax.ShapeDtypeStruct((B, 3, D), f32),
        ],
        compiler_params=pltpu.CompilerParams(
            dimension_semantics=("parallel",)),
    )(ze, cbt, codebooks, code_table)

    zq = jnp.concatenate([quant, cemb], axis=-1)
    zqp = jnp.pad(zq, ((0, 0), (0, 16), (0, 0)))
    qtp = jnp.pad(quant, ((0, 0), (0, 32), (0, 0)))
    c0p = jnp.pad(codes[:, :, 0:1], ((0, 0), (0, 32), (0, 0)))
    w3r = jnp.transpose(W3.reshape(DP, H, 2 * D), (1, 2, 0)) \
        .astype(jnp.bfloat16)
    w4t = W4.T.astype(jnp.bfloat16)

    gl = pl.pallas_call(
        _pred_kernel,
        grid=(B, NB),
        in_specs=[
            pl.BlockSpec((1, T + 16, 2 * D), lambda b, nb: (b, 0, 0)),
            pl.BlockSpec((1, T + 32, D), lambda b, nb: (b, 0, 0)),
            pl.BlockSpec((1, T + 32, 1), lambda b, nb: (b, 0, 0)),
            pl.BlockSpec((H, 2 * D, DP), lambda b, nb: (0, 0, 0)),
            pl.BlockSpec((1, DP), lambda b, nb: (0, 0)),
            pl.BlockSpec((DP, DP), lambda b, nb: (0, 0)),
            pl.BlockSpec((1, DP), lambda b, nb: (0, 0)),
        ],
        out_specs=pl.BlockSpec((1, 1, 1, 128), lambda b, nb: (b, nb, 0, 0)),
        out_shape=jax.ShapeDtypeStruct((B, NB, 1, 128), f32),
        compiler_params=pltpu.CompilerParams(
            dimension_semantics=("parallel", "parallel")),
    )(zqp, qtp, c0p, w3r, b3.reshape(1, DP), w4t, b4.reshape(1, DP))

    group_loss = jnp.sum(gl) / (B * N * P)
    vq0 = jnp.sum(misc[:, 0, :]) / (B * T * D)
    vq1 = jnp.sum(misc[:, 1, :]) / (B * T * D)
    vq_loss = 0.5 * (vq0 + vq1)
    smooth_loss = jnp.sum(misc[:, 2, :]) / (B * (T - 1))
    loss = group_loss * W_REC + vq_loss * W_COMMIT + smooth_loss * W_SMOOTH
    return codes, loss, group_loss, vq_loss, smooth_loss


# speedup vs baseline: 1.2953x; 1.0278x over previous
import jax
import jax.numpy as jnp
from jax.experimental import pallas as pl
from jax.experimental.pallas import tpu as pltpu

B, T = 8, 2048
INPUT_DIM = 2
HIDDEN = 256
D = 64
K = 1024
H, P = 10, 20
N = T - H - P + 1
DP = D * P
BN = 512
NB = T // BN
CH = 512
W_REC, W_COMMIT, W_SMOOTH = 1.0, 0.25, 0.1


def _shift_rows(a, off):
    if off == 0:
        return a
    L, C = a.shape
    z = jnp.zeros((abs(off), C), a.dtype)
    if off > 0:
        return jnp.concatenate([a[off:, :], z], axis=0)
    return jnp.concatenate([z, a[:off, :]], axis=0)


def _encoder_kernel(xt_ref, w1_ref, b1_ref, w2_ref, b2_ref, ze_ref):
    xt = xt_ref[0]
    z1 = jnp.zeros((T, HIDDEN), jnp.float32)
    for k in range(7):
        z1 = z1 + jnp.dot(_shift_rows(xt, k - 3), w1_ref[k],
                          preferred_element_type=jnp.float32)
    z1 = jnp.maximum(z1 + b1_ref[...], 0.0)
    z2 = jnp.zeros((T, D), jnp.float32)
    for k in range(9):
        z2 = z2 + jnp.dot(_shift_rows(z1, k - 4), w2_ref[k],
                          preferred_element_type=jnp.float32)
    ze_ref[0] = z2 + b2_ref[...]


def _split3(x):
    hi = x.astype(jnp.bfloat16)
    r1 = x - hi.astype(jnp.float32)
    mid = r1.astype(jnp.bfloat16)
    lo = (r1 - mid.astype(jnp.float32)).astype(jnp.bfloat16)
    return (hi, mid, lo)


def _vq_kernel(ze_ref, cbt_ref, cb_ref, ct_ref,
               quant_ref, zq_ref, codes_ref, misc_ref):
    l0 = jnp.zeros((1, D), jnp.float32)
    l1 = jnp.zeros((1, D), jnp.float32)
    idx0_chunks = []
    lane_iota = jax.lax.broadcasted_iota(jnp.int32, (CH, K), 1)

    cb0h, cb0m, cb0l = _split3(cb_ref[0])
    cb1h, cb1m, cb1l = _split3(cb_ref[1])
    cth, ctm, ctl = _split3(ct_ref[...])
    t0a = jnp.concatenate([cb0h, cb0m], axis=1)
    t0b = jnp.concatenate([cb0l, cth], axis=1)
    t0c = jnp.concatenate([ctm, ctl], axis=1)
    t1a = jnp.concatenate([cb1h, cb1m], axis=1)

    cbt0, cbt1 = cbt_ref[0], cbt_ref[1]
    cnorm0 = jnp.sum(cbt0 * cbt0, axis=0, keepdims=True)
    cnorm1 = jnp.sum(cbt1 * cbt1, axis=0, keepdims=True)

    def stage(r, cbt, cnorm):
        s = jnp.dot(r, cbt, preferred_element_type=jnp.float32)
        rnorm = jnp.sum(r * r, axis=1, keepdims=True)
        d2 = rnorm - 2.0 * s + cnorm
        dmin = jnp.min(d2, axis=1, keepdims=True)
        idx = jnp.min(jnp.where(d2 == dmin, lane_iota, K),
                      axis=1, keepdims=True)
        oh = (lane_iota == idx).astype(jnp.bfloat16)
        return idx, oh

    def dotf(oh, tab):
        return jnp.dot(oh, tab, preferred_element_type=jnp.float32)

    for c in range(T // CH):
        sl = slice(c * CH, (c + 1) * CH)
        z = ze_ref[0, sl, :]
        idx0, oh0 = stage(z, cbt0, cnorm0)
        p1 = dotf(oh0, t0a)
        p2 = dotf(oh0, t0b)
        p3 = dotf(oh0, t0c)
        qv0 = (p1[:, :D] + p1[:, D:]) + p2[:, :D]
        d0 = qv0 - z
        qst0 = z + d0
        r1 = z - qst0
        l0 = l0 + jnp.sum(d0 * d0, axis=0, keepdims=True)
        idx1, oh1 = stage(r1, cbt1, cnorm1)
        q1 = dotf(oh1, t1a)
        qv1 = (q1[:, :D] + q1[:, D:]) + dotf(oh1, cb1l)
        d1 = qv1 - r1
        qst1 = r1 + d1
        l1 = l1 + jnp.sum(d1 * d1, axis=0, keepdims=True)
        qsum = qst0 + qst1
        cemb = (p2[:, D:] + p3[:, :D]) + p3[:, D:]
        quant_ref[0, sl, :] = qsum
        zq_ref[0, sl, 0:D] = qsum
        zq_ref[0, sl, D:2 * D] = cemb
        codes_ref[0, sl, 0:1] = idx0
        codes_ref[0, sl, 1:2] = idx1
        idx0_chunks.append(idx0)

    quant_ref[0, T:, :] = jnp.zeros((32, D), jnp.float32)
    zq_ref[0, T:, :] = jnp.zeros((16, 2 * D), jnp.float32)

    idx0_full = jnp.concatenate(idx0_chunks, axis=0)
    tr = jnp.sum((idx0_full[1:, :] != idx0_full[:-1, :]).astype(jnp.float32),
                 axis=0, keepdims=True)
    trv = jnp.where(
        jax.lax.broadcasted_iota(jnp.int32, (1, D), 1) == 0, tr, 0.0)
    misc_ref[0, 0:1, :] = l0
    misc_ref[0, 1:2, :] = l1
    misc_ref[0, 2:3, :] = trv


def _pred_kernel(zq_ref, qt_ref, c0_ref, w3_ref, b3_ref, w4_ref, b4_ref,
                 gl_ref):
    nb = pl.program_id(1)
    n0 = nb * BN
    acc = jnp.zeros((BN, DP), jnp.float32)
    for i in range(H):
        acc = acc + jnp.dot(zq_ref[0, pl.ds(n0 + i, BN), :], w3_ref[i],
                            preferred_element_type=jnp.float32)
    h = jnp.maximum(acc + b3_ref[...], 0.0)
    zp = jnp.dot(h, w4_ref[...], preferred_element_type=jnp.float32) \
        + b4_ref[...]

    nvec = jax.lax.broadcasted_iota(jnp.int32, (BN, 1), 0) + n0
    mask = (nvec < N).astype(jnp.float32)
    ca = c0_ref[0, pl.ds(n0 + H, BN), :]
    cbn = c0_ref[0, pl.ds(n0 + H + 1, BN), :]
    sc = (ca == cbn).astype(jnp.float32)
    total = jnp.zeros((BN, 1), jnp.float32)
    for p in range(P):
        zt = qt_ref[0, pl.ds(n0 + H + p, BN), :]
        zpp = zp[:, p * D:(p + 1) * D]
        num = jnp.sum(zt * zpp, axis=1, keepdims=True)
        nt = jnp.sqrt(jnp.sum(zt * zt, axis=1, keepdims=True))
        npp = jnp.sqrt(jnp.sum(zpp * zpp, axis=1, keepdims=True))
        den = jnp.maximum(nt, 1e-8) * jnp.maximum(npp, 1e-8)
        sim = num / den
        dlt = sim - sc
        total = total + dlt * dlt
    tsum = jnp.sum(total * mask)
    gl_ref[0, 0] = jnp.where(
        jax.lax.broadcasted_iota(jnp.int32, (1, 128), 1) == 0, tsum, 0.0)


def kernel(traj_xy, masks, W1, b1, W2, b2, W3, b3, W4, b4, code_table,
           codebooks):
    f32 = jnp.float32
    xt = jnp.transpose(traj_xy, (0, 2, 1))
    w1 = jnp.transpose(W1, (2, 1, 0))
    w2 = jnp.transpose(W2, (2, 1, 0))

    ze = pl.pallas_call(
        _encoder_kernel,
        grid=(B,),
        in_specs=[
            pl.BlockSpec((1, T, INPUT_DIM), lambda b: (b, 0, 0)),
            pl.BlockSpec((7, INPUT_DIM, HIDDEN), lambda b: (0, 0, 0)),
            pl.BlockSpec((1, HIDDEN), lambda b: (0, 0)),
            pl.BlockSpec((9, HIDDEN, D), lambda b: (0, 0, 0)),
            pl.BlockSpec((1, D), lambda b: (0, 0)),
        ],
        out_specs=pl.BlockSpec((1, T, D), lambda b: (b, 0, 0)),
        out_shape=jax.ShapeDtypeStruct((B, T, D), f32),
        compiler_params=pltpu.CompilerParams(
            dimension_semantics=("parallel",)),
    )(xt, w1, b1.reshape(1, HIDDEN), w2, b2.reshape(1, D))

    cbt = jnp.transpose(codebooks, (0, 2, 1))
    qtp, zqp, codes, misc = pl.pallas_call(
        _vq_kernel,
        grid=(B,),
        in_specs=[
            pl.BlockSpec((1, T, D), lambda b: (b, 0, 0)),
            pl.BlockSpec((2, D, K), lambda b: (0, 0, 0)),
            pl.BlockSpec((2, K, D), lambda b: (0, 0, 0)),
            pl.BlockSpec((K, D), lambda b: (0, 0)),
        ],
        out_specs=[
            pl.BlockSpec((1, T + 32, D), lambda b: (b, 0, 0)),
            pl.BlockSpec((1, T + 16, 2 * D), lambda b: (b, 0, 0)),
            pl.BlockSpec((1, T, 2), lambda b: (b, 0, 0)),
            pl.BlockSpec((1, 3, D), lambda b: (b, 0, 0)),
        ],
        out_shape=[
            jax.ShapeDtypeStruct((B, T + 32, D), f32),
            jax.ShapeDtypeStruct((B, T + 16, 2 * D), f32),
            jax.ShapeDtypeStruct((B, T, 2), jnp.int32),
            jax.ShapeDtypeStruct((B, 3, D), f32),
        ],
        compiler_params=pltpu.CompilerParams(
            dimension_semantics=("parallel",)),
    )(ze, cbt, codebooks, code_table)

    c0p = jnp.pad(codes[:, :, 0:1], ((0, 0), (0, 32), (0, 0)))
    w3r = jnp.transpose(W3.reshape(DP, H, 2 * D), (1, 2, 0)) \
        .astype(jnp.bfloat16)
    w4t = W4.T.astype(jnp.bfloat16)

    gl = pl.pallas_call(
        _pred_kernel,
        grid=(B, NB),
        in_specs=[
            pl.BlockSpec((1, T + 16, 2 * D), lambda b, nb: (b, 0, 0)),
            pl.BlockSpec((1, T + 32, D), lambda b, nb: (b, 0, 0)),
            pl.BlockSpec((1, T + 32, 1), lambda b, nb: (b, 0, 0)),
            pl.BlockSpec((H, 2 * D, DP), lambda b, nb: (0, 0, 0)),
            pl.BlockSpec((1, DP), lambda b, nb: (0, 0)),
            pl.BlockSpec((DP, DP), lambda b, nb: (0, 0)),
            pl.BlockSpec((1, DP), lambda b, nb: (0, 0)),
        ],
        out_specs=pl.BlockSpec((1, 1, 1, 128), lambda b, nb: (b, nb, 0, 0)),
        out_shape=jax.ShapeDtypeStruct((B, NB, 1, 128), f32),
        compiler_params=pltpu.CompilerParams(
            dimension_semantics=("parallel", "parallel")),
    )(zqp, qtp, c0p, w3r, b3.reshape(1, DP), w4t, b4.reshape(1, DP))

    group_loss = jnp.sum(gl) / (B * N * P)
    vq0 = jnp.sum(misc[:, 0, :]) / (B * T * D)
    vq1 = jnp.sum(misc[:, 1, :]) / (B * T * D)
    vq_loss = 0.5 * (vq0 + vq1)
    smooth_loss = jnp.sum(misc[:, 2, :]) / (B * (T - 1))
    loss = group_loss * W_REC + vq_loss * W_COMMIT + smooth_loss * W_SMOOTH
    return codes, loss, group_loss, vq_loss, smooth_loss


# speedup vs baseline: 1.2960x; 1.0005x over previous
import jax
import jax.numpy as jnp
from jax.experimental import pallas as pl
from jax.experimental.pallas import tpu as pltpu

B, T = 8, 2048
INPUT_DIM = 2
HIDDEN = 256
D = 64
K = 1024
H, P = 10, 20
N = T - H - P + 1
DP = D * P
BN = 512
NB = T // BN
CH = 512
W_REC, W_COMMIT, W_SMOOTH = 1.0, 0.25, 0.1


def _shift_rows(a, off):
    if off == 0:
        return a
    L, C = a.shape
    z = jnp.zeros((abs(off), C), a.dtype)
    if off > 0:
        return jnp.concatenate([a[off:, :], z], axis=0)
    return jnp.concatenate([z, a[:off, :]], axis=0)


def _encoder_kernel(xt_ref, w1_ref, b1_ref, w2_ref, b2_ref, ze_ref):
    xt = xt_ref[0]
    z1 = jnp.zeros((T, HIDDEN), jnp.float32)
    for k in range(7):
        z1 = z1 + jnp.dot(_shift_rows(xt, k - 3), w1_ref[k],
                          preferred_element_type=jnp.float32)
    z1 = jnp.maximum(z1 + b1_ref[...], 0.0)
    z2 = jnp.zeros((T, D), jnp.float32)
    for k in range(9):
        z2 = z2 + jnp.dot(_shift_rows(z1, k - 4), w2_ref[k],
                          preferred_element_type=jnp.float32)
    ze_ref[0] = z2 + b2_ref[...]


def _split3(x):
    hi = x.astype(jnp.bfloat16)
    r1 = x - hi.astype(jnp.float32)
    mid = r1.astype(jnp.bfloat16)
    lo = (r1 - mid.astype(jnp.float32)).astype(jnp.bfloat16)
    return (hi, mid, lo)


def _vq_kernel(ze_ref, cbt_ref, cb_ref, ct_ref,
               quant_ref, zq_ref, codes_ref, misc_ref):
    l0 = jnp.zeros((1, D), jnp.float32)
    l1 = jnp.zeros((1, D), jnp.float32)
    idx0_chunks = []
    lane_iota = jax.lax.broadcasted_iota(jnp.int32, (CH, K), 1)

    cb0h, cb0m, cb0l = _split3(cb_ref[0])
    cb1h, cb1m, cb1l = _split3(cb_ref[1])
    cth, ctm, ctl = _split3(ct_ref[...])
    t0a = jnp.concatenate([cb0h, cb0m], axis=1)
    t0b = jnp.concatenate([cb0l, cth], axis=1)
    t0c = jnp.concatenate([ctm, ctl], axis=1)
    t1a = jnp.concatenate([cb1h, cb1m], axis=1)

    cbt0, cbt1 = cbt_ref[0], cbt_ref[1]
    cnorm0 = jnp.sum(cbt0 * cbt0, axis=0, keepdims=True)
    cnorm1 = jnp.sum(cbt1 * cbt1, axis=0, keepdims=True)

    def stage(r, cbt, cnorm):
        s = jnp.dot(r, cbt, preferred_element_type=jnp.float32)
        rnorm = jnp.sum(r * r, axis=1, keepdims=True)
        d2 = rnorm - 2.0 * s + cnorm
        dmin = jnp.min(d2, axis=1, keepdims=True)
        idx = jnp.min(jnp.where(d2 == dmin, lane_iota, K),
                      axis=1, keepdims=True)
        oh = (lane_iota == idx).astype(jnp.bfloat16)
        return idx, oh

    def dotf(oh, tab):
        return jnp.dot(oh, tab, preferred_element_type=jnp.float32)

    for c in range(T // CH):
        sl = slice(c * CH, (c + 1) * CH)
        z = ze_ref[0, sl, :]
        idx0, oh0 = stage(z, cbt0, cnorm0)
        p1 = dotf(oh0, t0a)
        p2 = dotf(oh0, t0b)
        p3 = dotf(oh0, t0c)
        qv0 = (p1[:, :D] + p1[:, D:]) + p2[:, :D]
        d0 = qv0 - z
        qst0 = z + d0
        r1 = z - qst0
        l0 = l0 + jnp.sum(d0 * d0, axis=0, keepdims=True)
        idx1, oh1 = stage(r1, cbt1, cnorm1)
        q1 = dotf(oh1, t1a)
        qv1 = (q1[:, :D] + q1[:, D:]) + dotf(oh1, cb1l)
        d1 = qv1 - r1
        qst1 = r1 + d1
        l1 = l1 + jnp.sum(d1 * d1, axis=0, keepdims=True)
        qsum = qst0 + qst1
        cemb = (p2[:, D:] + p3[:, :D]) + p3[:, D:]
        quant_ref[0, sl, :] = qsum
        zq_ref[0, sl, 0:D] = qsum
        zq_ref[0, sl, D:2 * D] = cemb
        codes_ref[0, sl, 0:1] = idx0
        codes_ref[0, sl, 1:2] = idx1
        idx0_chunks.append(idx0)

    quant_ref[0, T:, :] = jnp.zeros((32, D), jnp.float32)
    zq_ref[0, T:, :] = jnp.zeros((16, 2 * D), jnp.float32)

    idx0_full = jnp.concatenate(idx0_chunks, axis=0)
    tr = jnp.sum((idx0_full[1:, :] != idx0_full[:-1, :]).astype(jnp.float32),
                 axis=0, keepdims=True)
    trv = jnp.where(
        jax.lax.broadcasted_iota(jnp.int32, (1, D), 1) == 0, tr, 0.0)
    misc_ref[0, 0:1, :] = l0
    misc_ref[0, 1:2, :] = l1
    misc_ref[0, 2:3, :] = trv


def _pred_kernel(zq_ref, qt_ref, c0_ref, w3_ref, b3_ref, w4_ref, b4_ref,
                 gl_ref):
    nb = pl.program_id(1)
    n0 = nb * BN
    acc = jnp.zeros((BN, DP), jnp.float32)
    for i in range(H):
        acc = acc + jnp.dot(zq_ref[0, pl.ds(n0 + i, BN), :], w3_ref[i],
                            preferred_element_type=jnp.float32)
    h = jnp.maximum(acc + b3_ref[...], 0.0)
    zp = jnp.dot(h, w4_ref[...], preferred_element_type=jnp.float32) \
        + b4_ref[...]

    nvec = jax.lax.broadcasted_iota(jnp.int32, (BN, 1), 0) + n0
    mask = (nvec < N).astype(jnp.float32)
    ca = c0_ref[0, pl.ds(n0 + H, BN), :]
    cbn = c0_ref[0, pl.ds(n0 + H + 1, BN), :]
    sc = (ca == cbn).astype(jnp.float32)
    total = jnp.zeros((BN, 1), jnp.float32)
    for p in range(P):
        zt = qt_ref[0, pl.ds(n0 + H + p, BN), :]
        zpp = zp[:, p * D:(p + 1) * D]
        num = jnp.sum(zt * zpp, axis=1, keepdims=True)
        nt = jnp.sqrt(jnp.sum(zt * zt, axis=1, keepdims=True))
        npp = jnp.sqrt(jnp.sum(zpp * zpp, axis=1, keepdims=True))
        den = jnp.maximum(nt, 1e-8) * jnp.maximum(npp, 1e-8)
        sim = num / den
        dlt = sim - sc
        total = total + dlt * dlt
    tsum = jnp.sum(total * mask)
    gl_ref[0, 0] = jnp.where(
        jax.lax.broadcasted_iota(jnp.int32, (1, 128), 1) == 0, tsum, 0.0)


def kernel(traj_xy, masks, W1, b1, W2, b2, W3, b3, W4, b4, code_table,
           codebooks):
    f32 = jnp.float32
    xt = jnp.transpose(traj_xy, (0, 2, 1))
    w1 = jnp.transpose(W1, (2, 1, 0))
    w2 = jnp.transpose(W2, (2, 1, 0)).astype(jnp.bfloat16)

    ze = pl.pallas_call(
        _encoder_kernel,
        grid=(B,),
        in_specs=[
            pl.BlockSpec((1, T, INPUT_DIM), lambda b: (b, 0, 0)),
            pl.BlockSpec((7, INPUT_DIM, HIDDEN), lambda b: (0, 0, 0)),
            pl.BlockSpec((1, HIDDEN), lambda b: (0, 0)),
            pl.BlockSpec((9, HIDDEN, D), lambda b: (0, 0, 0)),
            pl.BlockSpec((1, D), lambda b: (0, 0)),
        ],
        out_specs=pl.BlockSpec((1, T, D), lambda b: (b, 0, 0)),
        out_shape=jax.ShapeDtypeStruct((B, T, D), f32),
        compiler_params=pltpu.CompilerParams(
            dimension_semantics=("parallel",)),
    )(xt, w1, b1.reshape(1, HIDDEN), w2, b2.reshape(1, D))

    cbt = jnp.transpose(codebooks, (0, 2, 1))
    qtp, zqp, codes, misc = pl.pallas_call(
        _vq_kernel,
        grid=(B,),
        in_specs=[
            pl.BlockSpec((1, T, D), lambda b: (b, 0, 0)),
            pl.BlockSpec((2, D, K), lambda b: (0, 0, 0)),
            pl.BlockSpec((2, K, D), lambda b: (0, 0, 0)),
            pl.BlockSpec((K, D), lambda b: (0, 0)),
        ],
        out_specs=[
            pl.BlockSpec((1, T + 32, D), lambda b: (b, 0, 0)),
            pl.BlockSpec((1, T + 16, 2 * D), lambda b: (b, 0, 0)),
            pl.BlockSpec((1, T, 2), lambda b: (b, 0, 0)),
            pl.BlockSpec((1, 3, D), lambda b: (b, 0, 0)),
        ],
        out_shape=[
            jax.ShapeDtypeStruct((B, T + 32, D), f32),
            jax.ShapeDtypeStruct((B, T + 16, 2 * D), f32),
            jax.ShapeDtypeStruct((B, T, 2), jnp.int32),
            jax.ShapeDtypeStruct((B, 3, D), f32),
        ],
        compiler_params=pltpu.CompilerParams(
            dimension_semantics=("parallel",)),
    )(ze, cbt, codebooks, code_table)

    c0p = jnp.pad(codes[:, :, 0:1], ((0, 0), (0, 32), (0, 0)))
    w3r = jnp.transpose(W3.reshape(DP, H, 2 * D), (1, 2, 0)) \
        .astype(jnp.bfloat16)
    w4t = W4.T.astype(jnp.bfloat16)

    gl = pl.pallas_call(
        _pred_kernel,
        grid=(B, NB),
        in_specs=[
            pl.BlockSpec((1, T + 16, 2 * D), lambda b, nb: (b, 0, 0)),
            pl.BlockSpec((1, T + 32, D), lambda b, nb: (b, 0, 0)),
            pl.BlockSpec((1, T + 32, 1), lambda b, nb: (b, 0, 0)),
            pl.BlockSpec((H, 2 * D, DP), lambda b, nb: (0, 0, 0)),
            pl.BlockSpec((1, DP), lambda b, nb: (0, 0)),
            pl.BlockSpec((DP, DP), lambda b, nb: (0, 0)),
            pl.BlockSpec((1, DP), lambda b, nb: (0, 0)),
        ],
        out_specs=pl.BlockSpec((1, 1, 1, 128), lambda b, nb: (b, nb, 0, 0)),
        out_shape=jax.ShapeDtypeStruct((B, NB, 1, 128), f32),
        compiler_params=pltpu.CompilerParams(
            dimension_semantics=("parallel", "parallel")),
    )(zqp, qtp, c0p, w3r, b3.reshape(1, DP), w4t, b4.reshape(1, DP))

    group_loss = jnp.sum(gl) / (B * N * P)
    vq0 = jnp.sum(misc[:, 0, :]) / (B * T * D)
    vq1 = jnp.sum(misc[:, 1, :]) / (B * T * D)
    vq_loss = 0.5 * (vq0 + vq1)
    smooth_loss = jnp.sum(misc[:, 2, :]) / (B * (T - 1))
    loss = group_loss * W_REC + vq_loss * W_COMMIT + smooth_loss * W_SMOOTH
    return codes, loss, group_loss, vq_loss, smooth_loss
